# baseline (speedup 1.0000x reference)
import functools

import jax
import jax.numpy as jnp
from jax import lax
from jax.experimental import pallas as pl
from jax.experimental.pallas import tpu as pltpu
from jax.experimental.pallas import tpu_sc as plsc

N_NODES = 10000
N_PAD = 10240
D = 128
C = 128
E = 320000
NC, NS = 2, 16
NW = NC * NS
CHUNK = 48
NCH = 210
E_PAD = (NCH + 3) * NW * CHUNK
ROWS_PER_TILE = N_PAD // NS
BLK = 1024


def _prep_body(x_ref, wt_ref, vs_ref, vd_ref, xs_ref, asrc_ref, adst_ref):
    xs = jnp.dot(x_ref[...], wt_ref[...], preferred_element_type=jnp.float32)
    xs_ref[...] = xs
    asrc_ref[...] = jnp.sum(xs * vs_ref[...][None, :], axis=1)
    adst_ref[...] = jnp.sum(xs * vd_ref[...][None, :], axis=1)


def _prep(x_pad, wt, vs, vd):
    grid = (N_PAD // BLK,)
    return pl.pallas_call(
        _prep_body,
        grid=grid,
        in_specs=[
            pl.BlockSpec((BLK, D), lambda i: (i, 0)),
            pl.BlockSpec((D, C), lambda i: (0, 0)),
            pl.BlockSpec((C,), lambda i: (0,)),
            pl.BlockSpec((C,), lambda i: (0,)),
        ],
        out_specs=[
            pl.BlockSpec((BLK, C), lambda i: (i, 0)),
            pl.BlockSpec((BLK,), lambda i: (i,)),
            pl.BlockSpec((BLK,), lambda i: (i,)),
        ],
        out_shape=[
            jax.ShapeDtypeStruct((N_PAD, C), jnp.float32),
            jax.ShapeDtypeStruct((N_PAD,), jnp.float32),
            jax.ShapeDtypeStruct((N_PAD,), jnp.float32),
        ],
    )(x_pad, wt, vs, vd)


def _sc_body(xs_hbm, asrc_hbm, adst_hbm, edges_hbm,
             acc_out, den_out,
             asrc_v, adst_v, idx0_v, idx1_v, idx2_v, idx3_v,
             ds0_v, ds1_v, e0_v, e1_v,
             rows0_v, rows1_v, rows2_v, rows3_v, acc_sh, den_sh,
             sem_g0, sem_g1, sem_g2, sem_g3, sem_s0, sem_s1):
    cid = lax.axis_index("c")
    sid = lax.axis_index("s")
    wid = cid * NS + sid
    idx2 = (idx0_v, idx1_v, idx2_v, idx3_v)
    dsv = (ds0_v, ds1_v)
    ev = (e0_v, e1_v)
    rows = (rows0_v, rows1_v, rows2_v, rows3_v)
    sem_g = (sem_g0, sem_g1, sem_g2, sem_g3)
    sem_s = (sem_s0, sem_s1)

    pltpu.sync_copy(asrc_hbm, asrc_v)
    pltpu.sync_copy(adst_hbm, adst_v)

    zeros16 = jnp.zeros((16,), jnp.float32)

    def _zrow(i, carry):
        for q in range(C // 16):
            rows0_v[i, pl.ds(q * 16, 16)] = zeros16
        return carry

    lax.fori_loop(0, CHUNK, _zrow, 0)
    for q in range(CHUNK // 16):
        e0_v[pl.ds(q * 16, 16)] = zeros16
    base_r = sid * ROWS_PER_TILE
    nfull = ROWS_PER_TILE // CHUNK
    rest = ROWS_PER_TILE - nfull * CHUNK
    for t in range(nfull):
        pltpu.sync_copy(rows0_v, acc_sh.at[pl.ds(base_r + t * CHUNK, CHUNK)])
        pltpu.sync_copy(e0_v, den_sh.at[pl.ds(base_r + t * CHUNK, CHUNK)])
    pltpu.sync_copy(rows0_v.at[pl.ds(0, rest)],
                    acc_sh.at[pl.ds(base_r + nfull * CHUNK, rest)])
    pltpu.sync_copy(e0_v.at[pl.ds(0, rest)],
                    den_sh.at[pl.ds(base_r + nfull * CHUNK, rest)])
    plsc.subcore_barrier()

    def _issue_gather(b4, g):
        base = (wid + NW * g) * CHUNK
        pltpu.sync_copy(edges_hbm.at[:, pl.ds(base, CHUNK)], idx2[b4])
        pltpu.async_copy(xs_hbm.at[idx2[b4].at[0]], rows[b4], sem_g[b4])

    def _wait_gather(b4):
        pltpu.make_async_copy(
            xs_hbm.at[idx2[b4].at[0]], rows[b4], sem_g[b4]).wait()

    def _wait_scatter(b4, b2):
        pltpu.make_async_copy(
            rows[b4], acc_sh.at[dsv[b2]], sem_s[b2]).wait()
        pltpu.make_async_copy(
            ev[b2], den_sh.at[dsv[b2]], sem_s[b2]).wait()

    def _process(b4, b2):
        for g in range(CHUNK // 16):
            sl = pl.ds(g * 16, 16)
            si = idx2[b4][0, sl]
            di = idx2[b4][1, sl]
            s = plsc.load_gather(asrc_v, [si]) + plsc.load_gather(adst_v, [di])
            s = jnp.where(s >= 0.0, s, 0.2 * s)
            ev[b2][sl] = jnp.where(si != di, jnp.exp(s), 0.0)
            dsv[b2][sl] = di

        def _scale(it, carry):
            for u in range(4):
                i = it * 4 + u
                spl = plsc.load_gather(
                    ev[b2], [jnp.full((16,), 0, jnp.int32) + i])
                for q in range(C // 16):
                    sl = pl.ds(q * 16, 16)
                    rows[b4][i, sl] = rows[b4][i, sl] * spl
            return carry

        lax.fori_loop(0, CHUNK // 4, _scale, 0)

    def _issue_scatter(b4, b2):
        pltpu.async_copy(rows[b4], acc_sh.at[dsv[b2]], sem_s[b2], add=True)
        pltpu.async_copy(ev[b2], den_sh.at[dsv[b2]], sem_s[b2], add=True)

    def _iter(b4, b2, g, wait_s):
        if wait_s:
            _wait_scatter((b4 + 2) % 4, b2)
        _issue_gather((b4 + 3) % 4, g + 3)
        _wait_gather(b4)
        _process(b4, b2)
        _issue_scatter(b4, b2)

    _issue_gather(0, 0)
    _issue_gather(1, 1)
    _issue_gather(2, 2)
    _iter(0, 0, 0, wait_s=False)
    _iter(1, 1, 1, wait_s=False)

    def _quad(jj, carry):
        g = 4 * jj + 2
        _iter(2, 0, g, wait_s=True)
        _iter(3, 1, g + 1, wait_s=True)
        _iter(0, 0, g + 2, wait_s=True)
        _iter(1, 1, g + 3, wait_s=True)
        return carry

    lax.fori_loop(0, (NCH - 2) // 4, _quad, 0)
    _wait_gather(2)
    _wait_gather(3)
    _wait_gather(0)
    _wait_scatter((NCH - 2) % 4, 0)
    _wait_scatter((NCH - 1) % 4, 1)
    plsc.subcore_barrier()

    for t in range(nfull):
        r0 = base_r + t * CHUNK
        rv = rows[t % 4]
        pltpu.sync_copy(acc_sh.at[pl.ds(r0, CHUNK)], rv)
        pltpu.sync_copy(rv, acc_out.at[cid, pl.ds(r0, CHUNK)])
    r0 = base_r + nfull * CHUNK
    pltpu.sync_copy(acc_sh.at[pl.ds(r0, rest)], rows3_v.at[pl.ds(0, rest)])
    pltpu.sync_copy(rows3_v.at[pl.ds(0, rest)], acc_out.at[cid, pl.ds(r0, rest)])
    for t in range(ROWS_PER_TILE // C):
        r0 = base_r + t * C
        pltpu.sync_copy(den_sh.at[pl.ds(r0, C)], rows0_v.at[0])
        pltpu.sync_copy(rows0_v.at[0], den_out.at[cid, pl.ds(r0, C)])


_sc_edges = functools.partial(
    pl.kernel,
    out_type=[
        jax.ShapeDtypeStruct((NC, N_PAD, C), jnp.float32),
        jax.ShapeDtypeStruct((NC, N_PAD), jnp.float32),
    ],
    mesh=plsc.VectorSubcoreMesh(core_axis_name="c", subcore_axis_name="s"),
    compiler_params=pltpu.CompilerParams(
        needs_layout_passes=False, use_tc_tiling_on_sc=False),
    scratch_types=[
        pltpu.VMEM((N_PAD,), jnp.float32),
        pltpu.VMEM((N_PAD,), jnp.float32),
        pltpu.VMEM((2, CHUNK), jnp.int32),
        pltpu.VMEM((2, CHUNK), jnp.int32),
        pltpu.VMEM((2, CHUNK), jnp.int32),
        pltpu.VMEM((2, CHUNK), jnp.int32),
        pltpu.VMEM((CHUNK,), jnp.int32),
        pltpu.VMEM((CHUNK,), jnp.int32),
        pltpu.VMEM((CHUNK,), jnp.float32),
        pltpu.VMEM((CHUNK,), jnp.float32),
        pltpu.VMEM((CHUNK, C), jnp.float32),
        pltpu.VMEM((CHUNK, C), jnp.float32),
        pltpu.VMEM((CHUNK, C), jnp.float32),
        pltpu.VMEM((CHUNK, C), jnp.float32),
        pltpu.VMEM_SHARED((N_PAD, C), jnp.float32),
        pltpu.VMEM_SHARED((N_PAD,), jnp.float32),
        pltpu.SemaphoreType.DMA,
        pltpu.SemaphoreType.DMA,
        pltpu.SemaphoreType.DMA,
        pltpu.SemaphoreType.DMA,
        pltpu.SemaphoreType.DMA,
        pltpu.SemaphoreType.DMA,
    ],
)(_sc_body)


def _combine_body(acc0_ref, acc1_ref, den0_ref, den1_ref, asrc_ref, adst_ref,
                  xs_ref, bias_ref, out_ref):
    s = asrc_ref[...] + adst_ref[...]
    s = jnp.where(s >= 0.0, s, 0.2 * s)
    es = jnp.exp(s)
    den = den0_ref[...] + den1_ref[...] + es
    num = acc0_ref[...] + acc1_ref[...] + es[:, None] * xs_ref[...]
    out_ref[...] = num / (den[:, None] + 1e-16) + bias_ref[...][None, :]


def _combine(acc0, acc1, den0, den1, asrc, adst, xs, bias):
    grid = (N_PAD // BLK,)
    mat = pl.BlockSpec((BLK, C), lambda i: (i, 0))
    vec = pl.BlockSpec((BLK,), lambda i: (i,))
    return pl.pallas_call(
        _combine_body,
        grid=grid,
        in_specs=[mat, mat, vec, vec, vec, vec, mat,
                  pl.BlockSpec((C,), lambda i: (0,))],
        out_specs=mat,
        out_shape=jax.ShapeDtypeStruct((N_PAD, C), jnp.float32),
    )(acc0, acc1, den0, den1, asrc, adst, xs, bias)


def kernel(x, edge_index, idx, W, att_src, att_dst, bias):
    sign = jnp.where(idx == 1, jnp.float32(-1.0), jnp.float32(1.0))
    vs = (sign * att_src).reshape(C).astype(jnp.float32)
    vd = (sign * att_dst).reshape(C).astype(jnp.float32)
    x_pad = jnp.concatenate(
        [x, jnp.zeros((N_PAD - N_NODES, D), jnp.float32)], axis=0)
    edges = jnp.concatenate(
        [edge_index, jnp.zeros((2, E_PAD - E), edge_index.dtype)], axis=1)

    xs, asrc, adst = _prep(x_pad, W.T, vs, vd)
    acc, den = _sc_edges(xs, asrc, adst, edges)
    out = _combine(acc[0], acc[1], den[0], den[1], asrc, adst, xs, bias)
    return out[:N_NODES]


# speedup vs baseline: 1.0296x; 1.0296x over previous
import functools

import jax
import jax.numpy as jnp
from jax import lax
from jax.experimental import pallas as pl
from jax.experimental.pallas import tpu as pltpu
from jax.experimental.pallas import tpu_sc as plsc

N_NODES = 10000
N_PAD = 10240
N_ACC = 10112
D = 128
C = 128
E = 320000
NC, NS = 2, 16
NW = NC * NS
CHUNK = 64
NCH = 158
E_PAD = (NCH + 1) * NW * CHUNK
ROWS_PER_TILE = N_ACC // NS
BLK = 1024


def _prep_body(x_ref, wt_ref, vs_ref, vd_ref, xs_ref, asrc_ref, adst_ref):
    xs = jnp.dot(x_ref[...], wt_ref[...], preferred_element_type=jnp.float32)
    xs_ref[...] = xs
    asrc_ref[...] = jnp.sum(xs * vs_ref[...][None, :], axis=1)
    adst_ref[...] = jnp.sum(xs * vd_ref[...][None, :], axis=1)


def _prep(x_pad, wt, vs, vd):
    grid = (N_PAD // BLK,)
    return pl.pallas_call(
        _prep_body,
        grid=grid,
        in_specs=[
            pl.BlockSpec((BLK, D), lambda i: (i, 0)),
            pl.BlockSpec((D, C), lambda i: (0, 0)),
            pl.BlockSpec((C,), lambda i: (0,)),
            pl.BlockSpec((C,), lambda i: (0,)),
        ],
        out_specs=[
            pl.BlockSpec((BLK, C), lambda i: (i, 0)),
            pl.BlockSpec((BLK,), lambda i: (i,)),
            pl.BlockSpec((BLK,), lambda i: (i,)),
        ],
        out_shape=[
            jax.ShapeDtypeStruct((N_PAD, C), jnp.float32),
            jax.ShapeDtypeStruct((N_PAD,), jnp.float32),
            jax.ShapeDtypeStruct((N_PAD,), jnp.float32),
        ],
    )(x_pad, wt, vs, vd)


def _sc_body(xsb_hbm, asrc_hbm, adst_hbm, edges_hbm,
             acc_out, den_out,
             asrc_v, adst_v, idx0_v, idx1_v, idx2_v,
             ds0_v, ds1_v, e_v,
             rb0_v, rb1_v, rb2_v, sc0_v, sc1_v, acc_sh, den_sh,
             sem_g0, sem_g1, sem_g2, sem_s0, sem_s1):
    cid = lax.axis_index("c")
    sid = lax.axis_index("s")
    wid = cid * NS + sid
    idx2 = (idx0_v, idx1_v, idx2_v)
    dsv = (ds0_v, ds1_v)
    rowsb = (rb0_v, rb1_v, rb2_v)
    scl = (sc0_v, sc1_v)
    sem_g = (sem_g0, sem_g1, sem_g2)
    sem_s = (sem_s0, sem_s1)

    pltpu.sync_copy(asrc_hbm.at[pl.ds(0, N_ACC)], asrc_v)
    pltpu.sync_copy(adst_hbm.at[pl.ds(0, N_ACC)], adst_v)

    zeros16 = jnp.zeros((16,), jnp.float32)

    def _zrow(i, carry):
        for q in range(C // 16):
            sc0_v[i, pl.ds(q * 16, 16)] = zeros16
        return carry

    lax.fori_loop(0, CHUNK, _zrow, 0)
    for q in range(CHUNK // 16):
        e_v[pl.ds(q * 16, 16)] = zeros16
    base_r = sid * ROWS_PER_TILE
    nfull = ROWS_PER_TILE // CHUNK
    rest = ROWS_PER_TILE - nfull * CHUNK
    for t in range(nfull):
        pltpu.sync_copy(sc0_v, acc_sh.at[pl.ds(base_r + t * CHUNK, CHUNK)])
        pltpu.sync_copy(e_v, den_sh.at[pl.ds(base_r + t * CHUNK, CHUNK)])
    pltpu.sync_copy(sc0_v.at[pl.ds(0, rest)],
                    acc_sh.at[pl.ds(base_r + nfull * CHUNK, rest)])
    pltpu.sync_copy(e_v.at[pl.ds(0, rest)],
                    den_sh.at[pl.ds(base_r + nfull * CHUNK, rest)])
    plsc.subcore_barrier()

    def _issue_gather(b3, g):
        base = (wid + NW * g) * CHUNK
        pltpu.sync_copy(edges_hbm.at[:, pl.ds(base, CHUNK)], idx2[b3])
        pltpu.async_copy(xsb_hbm.at[idx2[b3].at[0]], rowsb[b3], sem_g[b3])

    def _wait_gather(b3):
        pltpu.make_async_copy(
            xsb_hbm.at[idx2[b3].at[0]], rowsb[b3], sem_g[b3]).wait()

    def _wait_scatter(b2):
        pltpu.make_async_copy(
            scl[b2], acc_sh.at[dsv[b2]], sem_s[b2]).wait()

    def _process(b3, b2):
        for g in range(CHUNK // 16):
            sl = pl.ds(g * 16, 16)
            si = idx2[b3][0, sl]
            di = idx2[b3][1, sl]
            s = plsc.load_gather(asrc_v, [si]) + plsc.load_gather(adst_v, [di])
            s = jnp.where(s >= 0.0, s, 0.2 * s)
            e_v[sl] = jnp.where(si != di, jnp.exp(s), 0.0)
            dsv[b2][sl] = di

        def _scale(it, carry):
            for u in range(2):
                i = it * 2 + u
                spl = plsc.load_gather(
                    e_v, [jnp.full((16,), 0, jnp.int32) + i])
                for q in range(C // 32):
                    half = rowsb[b3][i, pl.ds(q * 32, 32)]
                    a, b = plsc.unpack(half, format=plsc.PackFormat.INTERLEAVED)
                    sc = scl[b2]
                    sc[i, pl.ds(q * 32, 16)] = a * spl
                    sc[i, pl.ds(q * 32 + 16, 16)] = b * spl
            return carry

        lax.fori_loop(0, CHUNK // 2, _scale, 0)

    def _iter(b3, b2, g, wait_s):
        if wait_s:
            _wait_scatter(b2)
        _issue_gather((b3 + 1) % 3, g + 1)
        _wait_gather(b3)
        _process(b3, b2)
        pltpu.async_copy(scl[b2], acc_sh.at[dsv[b2]], sem_s[b2], add=True)
        pltpu.sync_copy(e_v, den_sh.at[dsv[b2]], add=True)

    _issue_gather(0, 0)
    _iter(0, 0, 0, wait_s=False)
    _iter(1, 1, 1, wait_s=False)

    def _hex(jj, carry):
        g = 6 * jj + 2
        _iter(2, 0, g, wait_s=True)
        _iter(0, 1, g + 1, wait_s=True)
        _iter(1, 0, g + 2, wait_s=True)
        _iter(2, 1, g + 3, wait_s=True)
        _iter(0, 0, g + 4, wait_s=True)
        _iter(1, 1, g + 5, wait_s=True)
        return carry

    lax.fori_loop(0, (NCH - 2) // 6, _hex, 0)
    _wait_gather(NCH % 3)
    _wait_scatter(0)
    _wait_scatter(1)
    plsc.subcore_barrier()

    for t in range(nfull):
        r0 = base_r + t * CHUNK
        sv = scl[t % 2]
        pltpu.sync_copy(acc_sh.at[pl.ds(r0, CHUNK)], sv)
        pltpu.sync_copy(sv, acc_out.at[cid, pl.ds(r0, CHUNK)])
    r0 = base_r + nfull * CHUNK
    pltpu.sync_copy(acc_sh.at[pl.ds(r0, rest)], sc1_v.at[pl.ds(0, rest)])
    pltpu.sync_copy(sc1_v.at[pl.ds(0, rest)], acc_out.at[cid, pl.ds(r0, rest)])
    nden = ROWS_PER_TILE // C
    dres = ROWS_PER_TILE - nden * C
    for t in range(nden):
        r0 = base_r + t * C
        pltpu.sync_copy(den_sh.at[pl.ds(r0, C)], sc0_v.at[0])
        pltpu.sync_copy(sc0_v.at[0], den_out.at[cid, pl.ds(r0, C)])
    r0 = base_r + nden * C
    pltpu.sync_copy(den_sh.at[pl.ds(r0, dres)], sc0_v.at[0, pl.ds(0, dres)])
    pltpu.sync_copy(sc0_v.at[0, pl.ds(0, dres)], den_out.at[cid, pl.ds(r0, dres)])


_sc_edges = functools.partial(
    pl.kernel,
    out_type=[
        jax.ShapeDtypeStruct((NC, N_ACC, C), jnp.float32),
        jax.ShapeDtypeStruct((NC, N_ACC), jnp.float32),
    ],
    mesh=plsc.VectorSubcoreMesh(core_axis_name="c", subcore_axis_name="s"),
    compiler_params=pltpu.CompilerParams(
        needs_layout_passes=False, use_tc_tiling_on_sc=False),
    scratch_types=[
        pltpu.VMEM((N_ACC,), jnp.float32),
        pltpu.VMEM((N_ACC,), jnp.float32),
        pltpu.VMEM((2, CHUNK), jnp.int32),
        pltpu.VMEM((2, CHUNK), jnp.int32),
        pltpu.VMEM((2, CHUNK), jnp.int32),
        pltpu.VMEM((CHUNK,), jnp.int32),
        pltpu.VMEM((CHUNK,), jnp.int32),
        pltpu.VMEM((CHUNK,), jnp.float32),
        pltpu.VMEM((CHUNK, C), jnp.bfloat16),
        pltpu.VMEM((CHUNK, C), jnp.bfloat16),
        pltpu.VMEM((CHUNK, C), jnp.bfloat16),
        pltpu.VMEM((CHUNK, C), jnp.float32),
        pltpu.VMEM((CHUNK, C), jnp.float32),
        pltpu.VMEM_SHARED((N_ACC, C), jnp.float32),
        pltpu.VMEM_SHARED((N_ACC,), jnp.float32),
        pltpu.SemaphoreType.DMA,
        pltpu.SemaphoreType.DMA,
        pltpu.SemaphoreType.DMA,
        pltpu.SemaphoreType.DMA,
        pltpu.SemaphoreType.DMA,
    ],
)(_sc_body)


def _combine_body(acc0_ref, acc1_ref, den0_ref, den1_ref, asrc_ref, adst_ref,
                  xs_ref, bias_ref, out_ref):
    s = asrc_ref[...] + adst_ref[...]
    s = jnp.where(s >= 0.0, s, 0.2 * s)
    es = jnp.exp(s)
    den = den0_ref[...] + den1_ref[...] + es
    num = acc0_ref[...] + acc1_ref[...] + es[:, None] * xs_ref[...]
    out_ref[...] = num / (den[:, None] + 1e-16) + bias_ref[...][None, :]


def _combine(acc0, acc1, den0, den1, asrc, adst, xs, bias):
    grid = (N_PAD // BLK,)
    mat = pl.BlockSpec((BLK, C), lambda i: (i, 0))
    vec = pl.BlockSpec((BLK,), lambda i: (i,))
    return pl.pallas_call(
        _combine_body,
        grid=grid,
        in_specs=[mat, mat, vec, vec, vec, vec, mat,
                  pl.BlockSpec((C,), lambda i: (0,))],
        out_specs=mat,
        out_shape=jax.ShapeDtypeStruct((N_PAD, C), jnp.float32),
    )(acc0, acc1, den0, den1, asrc, adst, xs, bias)


def kernel(x, edge_index, idx, W, att_src, att_dst, bias):
    sign = jnp.where(idx == 1, jnp.float32(-1.0), jnp.float32(1.0))
    vs = (sign * att_src).reshape(C).astype(jnp.float32)
    vd = (sign * att_dst).reshape(C).astype(jnp.float32)
    x_pad = jnp.concatenate(
        [x, jnp.zeros((N_PAD - N_NODES, D), jnp.float32)], axis=0)
    edges = jnp.concatenate(
        [edge_index, jnp.zeros((2, E_PAD - E), edge_index.dtype)], axis=1)

    xs, asrc, adst = _prep(x_pad, W.T, vs, vd)
    xsb = (xs.astype(jnp.bfloat16)
             .reshape(N_PAD, C // 32, 2, 16)
             .transpose(0, 1, 3, 2)
             .reshape(N_PAD, C))
    acc, den = _sc_edges(xsb, asrc, adst, edges)
    accp = jnp.zeros((NC, N_PAD - N_ACC, C), jnp.float32)
    denp = jnp.zeros((NC, N_PAD - N_ACC), jnp.float32)
    acc = jnp.concatenate([acc, accp], axis=1)
    den = jnp.concatenate([den, denp], axis=1)
    out = _combine(acc[0], acc[1], den[0], den[1], asrc, adst, xs, bias)
    return out[:N_NODES]


# speedup vs baseline: 1.0591x; 1.0286x over previous
import functools

import jax
import jax.numpy as jnp
from jax import lax
from jax.experimental import pallas as pl
from jax.experimental.pallas import tpu as pltpu
from jax.experimental.pallas import tpu_sc as plsc

N_NODES = 10000
N_PAD = 10240
N_ACC = 10112
N_TAB = 10000
D = 128
C = 128
E = 320000
NC, NS = 2, 16
NW = NC * NS
CHUNK = 112
NCH = 90
E_PAD = (NCH + 1) * NW * CHUNK
ROWS_PER_TILE = N_ACC // NS
BLK = 1024


def _prep_body(x_ref, wt_ref, vs_ref, vd_ref, xs_ref, asrc_ref, adst_ref):
    xs = jnp.dot(x_ref[...], wt_ref[...], preferred_element_type=jnp.float32)
    xs_ref[...] = xs
    asrc_ref[...] = jnp.sum(xs * vs_ref[...][None, :], axis=1)
    adst_ref[...] = jnp.sum(xs * vd_ref[...][None, :], axis=1)


def _prep(x_pad, wt, vs, vd):
    grid = (N_PAD // BLK,)
    return pl.pallas_call(
        _prep_body,
        grid=grid,
        in_specs=[
            pl.BlockSpec((BLK, D), lambda i: (i, 0)),
            pl.BlockSpec((D, C), lambda i: (0, 0)),
            pl.BlockSpec((C,), lambda i: (0,)),
            pl.BlockSpec((C,), lambda i: (0,)),
        ],
        out_specs=[
            pl.BlockSpec((BLK, C), lambda i: (i, 0)),
            pl.BlockSpec((BLK,), lambda i: (i,)),
            pl.BlockSpec((BLK,), lambda i: (i,)),
        ],
        out_shape=[
            jax.ShapeDtypeStruct((N_PAD, C), jnp.float32),
            jax.ShapeDtypeStruct((N_PAD,), jnp.float32),
            jax.ShapeDtypeStruct((N_PAD,), jnp.float32),
        ],
    )(x_pad, wt, vs, vd)


def _sc_body(xs_hbm, asrc_hbm, adst_hbm, edges_hbm,
             acc_out, den_out,
             asrc_v, adst_v, idx0_v, idx1_v, e_v,
             rows0_v, rows1_v, acc_sh, den_sh,
             sem_g0, sem_g1):
    cid = lax.axis_index("c")
    sid = lax.axis_index("s")
    wid = cid * NS + sid
    idx2 = (idx0_v, idx1_v)
    rows = (rows0_v, rows1_v)
    sem_g = (sem_g0, sem_g1)

    pltpu.sync_copy(asrc_hbm.at[pl.ds(0, N_TAB)], asrc_v)
    pltpu.sync_copy(adst_hbm.at[pl.ds(0, N_TAB)], adst_v)

    zeros16 = jnp.zeros((16,), jnp.float32)

    def _zrow(i, carry):
        for q in range(C // 16):
            rows0_v[i, pl.ds(q * 16, 16)] = zeros16
        return carry

    lax.fori_loop(0, CHUNK, _zrow, 0)
    for q in range(CHUNK // 16):
        e_v[pl.ds(q * 16, 16)] = zeros16
    base_r = sid * ROWS_PER_TILE
    nfull = ROWS_PER_TILE // CHUNK
    rest = ROWS_PER_TILE - nfull * CHUNK
    for t in range(nfull):
        pltpu.sync_copy(rows0_v, acc_sh.at[pl.ds(base_r + t * CHUNK, CHUNK)])
        pltpu.sync_copy(e_v, den_sh.at[pl.ds(base_r + t * CHUNK, CHUNK)])
    pltpu.sync_copy(rows0_v.at[pl.ds(0, rest)],
                    acc_sh.at[pl.ds(base_r + nfull * CHUNK, rest)])
    pltpu.sync_copy(e_v.at[pl.ds(0, rest)],
                    den_sh.at[pl.ds(base_r + nfull * CHUNK, rest)])
    plsc.subcore_barrier()

    def _issue_gather(b, g):
        base = (wid + NW * g) * CHUNK
        pltpu.sync_copy(edges_hbm.at[:, pl.ds(base, CHUNK)], idx2[b])
        pltpu.async_copy(xs_hbm.at[idx2[b].at[0]], rows[b], sem_g[b])

    def _wait_gather(b):
        pltpu.make_async_copy(
            xs_hbm.at[idx2[b].at[0]], rows[b], sem_g[b]).wait()

    def _process(b):
        for g in range(CHUNK // 16):
            sl = pl.ds(g * 16, 16)
            si = idx2[b][0, sl]
            di = idx2[b][1, sl]
            s = plsc.load_gather(asrc_v, [si]) + plsc.load_gather(adst_v, [di])
            s = jnp.where(s >= 0.0, s, 0.2 * s)
            e_v[sl] = jnp.where(si != di, jnp.exp(s), 0.0)

        def _scale(it, carry):
            for u in range(4):
                i = it * 4 + u
                spl = plsc.load_gather(
                    e_v, [jnp.full((16,), 0, jnp.int32) + i])
                for q in range(C // 16):
                    sl = pl.ds(q * 16, 16)
                    rows[b][i, sl] = rows[b][i, sl] * spl
            return carry

        lax.fori_loop(0, CHUNK // 4, _scale, 0)

    def _iter(b, g):
        _issue_gather(1 - b, g + 1)
        _wait_gather(b)
        _process(b)
        pltpu.sync_copy(rows[b], acc_sh.at[idx2[b].at[1]], add=True)
        pltpu.sync_copy(e_v, den_sh.at[idx2[b].at[1]], add=True)

    _issue_gather(0, 0)

    def _pair(jj, carry):
        _iter(0, 2 * jj)
        _iter(1, 2 * jj + 1)
        return carry

    lax.fori_loop(0, NCH // 2, _pair, 0)
    _wait_gather(0)
    plsc.subcore_barrier()

    for t in range(nfull):
        r0 = base_r + t * CHUNK
        rv = rows[t % 2]
        pltpu.sync_copy(acc_sh.at[pl.ds(r0, CHUNK)], rv)
        pltpu.sync_copy(rv, acc_out.at[cid, pl.ds(r0, CHUNK)])
    r0 = base_r + nfull * CHUNK
    pltpu.sync_copy(acc_sh.at[pl.ds(r0, rest)], rows1_v.at[pl.ds(0, rest)])
    pltpu.sync_copy(rows1_v.at[pl.ds(0, rest)], acc_out.at[cid, pl.ds(r0, rest)])
    nden = ROWS_PER_TILE // C
    dres = ROWS_PER_TILE - nden * C
    for t in range(nden):
        r0 = base_r + t * C
        pltpu.sync_copy(den_sh.at[pl.ds(r0, C)], rows0_v.at[0])
        pltpu.sync_copy(rows0_v.at[0], den_out.at[cid, pl.ds(r0, C)])
    r0 = base_r + nden * C
    pltpu.sync_copy(den_sh.at[pl.ds(r0, dres)], rows0_v.at[0, pl.ds(0, dres)])
    pltpu.sync_copy(rows0_v.at[0, pl.ds(0, dres)],
                    den_out.at[cid, pl.ds(r0, dres)])


_sc_edges = functools.partial(
    pl.kernel,
    out_type=[
        jax.ShapeDtypeStruct((NC, N_ACC, C), jnp.float32),
        jax.ShapeDtypeStruct((NC, N_ACC), jnp.float32),
    ],
    mesh=plsc.VectorSubcoreMesh(core_axis_name="c", subcore_axis_name="s"),
    compiler_params=pltpu.CompilerParams(
        needs_layout_passes=False, use_tc_tiling_on_sc=False),
    scratch_types=[
        pltpu.VMEM((N_TAB,), jnp.float32),
        pltpu.VMEM((N_TAB,), jnp.float32),
        pltpu.VMEM((2, CHUNK), jnp.int32),
        pltpu.VMEM((2, CHUNK), jnp.int32),
        pltpu.VMEM((CHUNK,), jnp.float32),
        pltpu.VMEM((CHUNK, C), jnp.float32),
        pltpu.VMEM((CHUNK, C), jnp.float32),
        pltpu.VMEM_SHARED((N_ACC, C), jnp.float32),
        pltpu.VMEM_SHARED((N_ACC,), jnp.float32),
        pltpu.SemaphoreType.DMA,
        pltpu.SemaphoreType.DMA,
    ],
)(_sc_body)


def _combine_body(acc0_ref, acc1_ref, den0_ref, den1_ref, asrc_ref, adst_ref,
                  xs_ref, bias_ref, out_ref):
    s = asrc_ref[...] + adst_ref[...]
    s = jnp.where(s >= 0.0, s, 0.2 * s)
    es = jnp.exp(s)
    den = den0_ref[...] + den1_ref[...] + es
    num = acc0_ref[...] + acc1_ref[...] + es[:, None] * xs_ref[...]
    out_ref[...] = num / (den[:, None] + 1e-16) + bias_ref[...][None, :]


def _combine(acc0, acc1, den0, den1, asrc, adst, xs, bias):
    grid = (N_PAD // BLK,)
    mat = pl.BlockSpec((BLK, C), lambda i: (i, 0))
    vec = pl.BlockSpec((BLK,), lambda i: (i,))
    return pl.pallas_call(
        _combine_body,
        grid=grid,
        in_specs=[mat, mat, vec, vec, vec, vec, mat,
                  pl.BlockSpec((C,), lambda i: (0,))],
        out_specs=mat,
        out_shape=jax.ShapeDtypeStruct((N_PAD, C), jnp.float32),
    )(acc0, acc1, den0, den1, asrc, adst, xs, bias)


def kernel(x, edge_index, idx, W, att_src, att_dst, bias):
    sign = jnp.where(idx == 1, jnp.float32(-1.0), jnp.float32(1.0))
    vs = (sign * att_src).reshape(C).astype(jnp.float32)
    vd = (sign * att_dst).reshape(C).astype(jnp.float32)
    x_pad = jnp.concatenate(
        [x, jnp.zeros((N_PAD - N_NODES, D), jnp.float32)], axis=0)
    edges = jnp.concatenate(
        [edge_index, jnp.zeros((2, E_PAD - E), edge_index.dtype)], axis=1)

    xs, asrc, adst = _prep(x_pad, W.T, vs, vd)
    acc, den = _sc_edges(xs, asrc, adst, edges)
    accp = jnp.zeros((NC, N_PAD - N_ACC, C), jnp.float32)
    denp = jnp.zeros((NC, N_PAD - N_ACC), jnp.float32)
    acc = jnp.concatenate([acc, accp], axis=1)
    den = jnp.concatenate([den, denp], axis=1)
    out = _combine(acc[0], acc[1], den[0], den[1], asrc, adst, xs, bias)
    return out[:N_NODES]


# speedup vs baseline: 1.1779x; 1.1121x over previous
import functools

import jax
import jax.numpy as jnp
from jax import lax
from jax.experimental import pallas as pl
from jax.experimental.pallas import tpu as pltpu
from jax.experimental.pallas import tpu_sc as plsc

N_NODES = 10000
N_PAD = 10240
N_ACC = 10112
N_TAB = 10000
D = 128
C = 128
E = 320000
NC, NS = 2, 16
NW = NC * NS
CHUNK = 64
NCH = 158
E_PAD = (NCH + 1) * NW * CHUNK
ROWS_PER_TILE = N_ACC // NS
BLK = 1024


def _prep_body(x_ref, wt_ref, vs_ref, vd_ref, xs_ref, asrc_ref, adst_ref):
    xs = jnp.dot(x_ref[...], wt_ref[...], preferred_element_type=jnp.float32)
    xs_ref[...] = xs
    asrc_ref[...] = jnp.sum(xs * vs_ref[...][None, :], axis=1)
    adst_ref[...] = jnp.sum(xs * vd_ref[...][None, :], axis=1)


def _prep(x_pad, wt, vs, vd):
    grid = (N_PAD // BLK,)
    return pl.pallas_call(
        _prep_body,
        grid=grid,
        in_specs=[
            pl.BlockSpec((BLK, D), lambda i: (i, 0)),
            pl.BlockSpec((D, C), lambda i: (0, 0)),
            pl.BlockSpec((C,), lambda i: (0,)),
            pl.BlockSpec((C,), lambda i: (0,)),
        ],
        out_specs=[
            pl.BlockSpec((BLK, C), lambda i: (i, 0)),
            pl.BlockSpec((BLK,), lambda i: (i,)),
            pl.BlockSpec((BLK,), lambda i: (i,)),
        ],
        out_shape=[
            jax.ShapeDtypeStruct((N_PAD, C), jnp.float32),
            jax.ShapeDtypeStruct((N_PAD,), jnp.float32),
            jax.ShapeDtypeStruct((N_PAD,), jnp.float32),
        ],
    )(x_pad, wt, vs, vd)


def _sc_body(xs_hbm, asrc_hbm, adst_hbm, edges_hbm,
             acc_out, den_out,
             asrc_v, adst_v, idx0_v, idx1_v, idx2_v, e0_v, e1_v, e2_v,
             rows0_v, rows1_v, rows2_v, acc_sh, den_sh,
             sem_g0, sem_g1, sem_g2, sem_s0, sem_s1, sem_s2):
    cid = lax.axis_index("c")
    sid = lax.axis_index("s")
    wid = cid * NS + sid
    idx2 = (idx0_v, idx1_v, idx2_v)
    ev = (e0_v, e1_v, e2_v)
    rows = (rows0_v, rows1_v, rows2_v)
    sem_g = (sem_g0, sem_g1, sem_g2)
    sem_s = (sem_s0, sem_s1, sem_s2)

    pltpu.sync_copy(asrc_hbm.at[pl.ds(0, N_TAB)], asrc_v)
    pltpu.sync_copy(adst_hbm.at[pl.ds(0, N_TAB)], adst_v)

    zeros16 = jnp.zeros((16,), jnp.float32)

    def _zrow(i, carry):
        for q in range(C // 16):
            rows0_v[i, pl.ds(q * 16, 16)] = zeros16
        return carry

    lax.fori_loop(0, CHUNK, _zrow, 0)
    for q in range(CHUNK // 16):
        e0_v[pl.ds(q * 16, 16)] = zeros16
    base_r = sid * ROWS_PER_TILE
    nfull = ROWS_PER_TILE // CHUNK
    rest = ROWS_PER_TILE - nfull * CHUNK
    for t in range(nfull):
        pltpu.sync_copy(rows0_v, acc_sh.at[pl.ds(base_r + t * CHUNK, CHUNK)])
        pltpu.sync_copy(e0_v, den_sh.at[pl.ds(base_r + t * CHUNK, CHUNK)])
    pltpu.sync_copy(rows0_v.at[pl.ds(0, rest)],
                    acc_sh.at[pl.ds(base_r + nfull * CHUNK, rest)])
    pltpu.sync_copy(e0_v.at[pl.ds(0, rest)],
                    den_sh.at[pl.ds(base_r + nfull * CHUNK, rest)])
    plsc.subcore_barrier()

    def _issue_gather(b, g):
        base = (wid + NW * g) * CHUNK
        pltpu.sync_copy(edges_hbm.at[:, pl.ds(base, CHUNK)], idx2[b])
        pltpu.async_copy(xs_hbm.at[idx2[b].at[0]], rows[b], sem_g[b])

    def _wait_gather(b):
        pltpu.make_async_copy(
            xs_hbm.at[idx2[b].at[0]], rows[b], sem_g[b]).wait()

    def _wait_scatter(b):
        pltpu.make_async_copy(
            rows[b], acc_sh.at[idx2[b].at[1]], sem_s[b]).wait()
        pltpu.make_async_copy(
            ev[b], den_sh.at[idx2[b].at[1]], sem_s[b]).wait()

    def _process(b):
        for g in range(CHUNK // 16):
            sl = pl.ds(g * 16, 16)
            si = idx2[b][0, sl]
            di = idx2[b][1, sl]
            s = plsc.load_gather(asrc_v, [si]) + plsc.load_gather(adst_v, [di])
            s = jnp.where(s >= 0.0, s, 0.2 * s)
            ev[b][sl] = jnp.where(si != di, jnp.exp(s), 0.0)

        def _scale(it, carry):
            for u in range(4):
                i = it * 4 + u
                spl = plsc.load_gather(
                    ev[b], [jnp.full((16,), 0, jnp.int32) + i])
                for q in range(C // 16):
                    sl = pl.ds(q * 16, 16)
                    rows[b][i, sl] = rows[b][i, sl] * spl
            return carry

        lax.fori_loop(0, CHUNK // 4, _scale, 0)

    def _issue_scatter(b):
        pltpu.async_copy(rows[b], acc_sh.at[idx2[b].at[1]], sem_s[b], add=True)
        pltpu.async_copy(ev[b], den_sh.at[idx2[b].at[1]], sem_s[b], add=True)

    def _iter(b, g, wait_s):
        nb = (b + 1) % 3
        if wait_s:
            _wait_scatter(nb)
        _issue_gather(nb, g + 1)
        _wait_gather(b)
        _process(b)
        _issue_scatter(b)

    _issue_gather(0, 0)
    _iter(0, 0, wait_s=False)
    _iter(1, 1, wait_s=False)

    def _trip(jj, carry):
        _iter(2, 3 * jj + 2, wait_s=True)
        _iter(0, 3 * jj + 3, wait_s=True)
        _iter(1, 3 * jj + 4, wait_s=True)
        return carry

    lax.fori_loop(0, (NCH - 5) // 3 + 1, _trip, 0)
    _wait_gather(2)
    _wait_scatter(0)
    _wait_scatter(1)
    plsc.subcore_barrier()

    for t in range(nfull):
        r0 = base_r + t * CHUNK
        rv = rows[t % 3]
        pltpu.sync_copy(acc_sh.at[pl.ds(r0, CHUNK)], rv)
        pltpu.sync_copy(rv, acc_out.at[cid, pl.ds(r0, CHUNK)])
    r0 = base_r + nfull * CHUNK
    pltpu.sync_copy(acc_sh.at[pl.ds(r0, rest)], rows1_v.at[pl.ds(0, rest)])
    pltpu.sync_copy(rows1_v.at[pl.ds(0, rest)], acc_out.at[cid, pl.ds(r0, rest)])
    nden = ROWS_PER_TILE // C
    dres = ROWS_PER_TILE - nden * C
    for t in range(nden):
        r0 = base_r + t * C
        pltpu.sync_copy(den_sh.at[pl.ds(r0, C)], rows0_v.at[0])
        pltpu.sync_copy(rows0_v.at[0], den_out.at[cid, pl.ds(r0, C)])
    r0 = base_r + nden * C
    pltpu.sync_copy(den_sh.at[pl.ds(r0, dres)], rows0_v.at[0, pl.ds(0, dres)])
    pltpu.sync_copy(rows0_v.at[0, pl.ds(0, dres)],
                    den_out.at[cid, pl.ds(r0, dres)])


_sc_edges = functools.partial(
    pl.kernel,
    out_type=[
        jax.ShapeDtypeStruct((NC, N_ACC, C), jnp.float32),
        jax.ShapeDtypeStruct((NC, N_ACC), jnp.float32),
    ],
    mesh=plsc.VectorSubcoreMesh(core_axis_name="c", subcore_axis_name="s"),
    compiler_params=pltpu.CompilerParams(
        needs_layout_passes=False, use_tc_tiling_on_sc=False),
    scratch_types=[
        pltpu.VMEM((N_TAB,), jnp.float32),
        pltpu.VMEM((N_TAB,), jnp.float32),
        pltpu.VMEM((2, CHUNK), jnp.int32),
        pltpu.VMEM((2, CHUNK), jnp.int32),
        pltpu.VMEM((2, CHUNK), jnp.int32),
        pltpu.VMEM((CHUNK,), jnp.float32),
        pltpu.VMEM((CHUNK,), jnp.float32),
        pltpu.VMEM((CHUNK,), jnp.float32),
        pltpu.VMEM((CHUNK, C), jnp.float32),
        pltpu.VMEM((CHUNK, C), jnp.float32),
        pltpu.VMEM((CHUNK, C), jnp.float32),
        pltpu.VMEM_SHARED((N_ACC, C), jnp.float32),
        pltpu.VMEM_SHARED((N_ACC,), jnp.float32),
        pltpu.SemaphoreType.DMA,
        pltpu.SemaphoreType.DMA,
        pltpu.SemaphoreType.DMA,
        pltpu.SemaphoreType.DMA,
        pltpu.SemaphoreType.DMA,
        pltpu.SemaphoreType.DMA,
    ],
)(_sc_body)


def _combine_body(acc0_ref, acc1_ref, den0_ref, den1_ref, asrc_ref, adst_ref,
                  xs_ref, bias_ref, out_ref):
    s = asrc_ref[...] + adst_ref[...]
    s = jnp.where(s >= 0.0, s, 0.2 * s)
    es = jnp.exp(s)
    den = den0_ref[...] + den1_ref[...] + es
    num = acc0_ref[...] + acc1_ref[...] + es[:, None] * xs_ref[...]
    out_ref[...] = num / (den[:, None] + 1e-16) + bias_ref[...][None, :]


def _combine(acc0, acc1, den0, den1, asrc, adst, xs, bias):
    grid = (N_PAD // BLK,)
    mat = pl.BlockSpec((BLK, C), lambda i: (i, 0))
    vec = pl.BlockSpec((BLK,), lambda i: (i,))
    return pl.pallas_call(
        _combine_body,
        grid=grid,
        in_specs=[mat, mat, vec, vec, vec, vec, mat,
                  pl.BlockSpec((C,), lambda i: (0,))],
        out_specs=mat,
        out_shape=jax.ShapeDtypeStruct((N_PAD, C), jnp.float32),
    )(acc0, acc1, den0, den1, asrc, adst, xs, bias)


def kernel(x, edge_index, idx, W, att_src, att_dst, bias):
    sign = jnp.where(idx == 1, jnp.float32(-1.0), jnp.float32(1.0))
    vs = (sign * att_src).reshape(C).astype(jnp.float32)
    vd = (sign * att_dst).reshape(C).astype(jnp.float32)
    x_pad = jnp.concatenate(
        [x, jnp.zeros((N_PAD - N_NODES, D), jnp.float32)], axis=0)
    edges = jnp.concatenate(
        [edge_index, jnp.zeros((2, E_PAD - E), edge_index.dtype)], axis=1)

    xs, asrc, adst = _prep(x_pad, W.T, vs, vd)
    acc, den = _sc_edges(xs, asrc, adst, edges)
    accp = jnp.zeros((NC, N_PAD - N_ACC, C), jnp.float32)
    denp = jnp.zeros((NC, N_PAD - N_ACC), jnp.float32)
    acc = jnp.concatenate([acc, accp], axis=1)
    den = jnp.concatenate([den, denp], axis=1)
    out = _combine(acc[0], acc[1], den[0], den[1], asrc, adst, xs, bias)
    return out[:N_NODES]
